# Initial kernel scaffold; baseline (speedup 1.0000x reference)
import jax
import jax.numpy as jnp
from jax.experimental import pallas as pl


def kernel(emb_table, w_enc_pad, b_enc_pad, w_c_pad, b_c_pad, w_sum_pad, b_sum_pad):
    raise NotImplementedError("write your pallas kernel here")


# speedup vs baseline: 1.8744x; 1.8744x over previous
import numpy as np
import jax
import jax.numpy as jnp
from jax.experimental import pallas as pl
from jax.experimental.pallas import tpu as pltpu

_B = 32
_N = 127
_T = 10
_FEAT = 128
_AGG = 32


def _build_schedule():
    order = [(0, b) for b in range(_B)]
    waves = [order]
    for _ in range(6):
        order = ([(2 * n + 1, b) for (n, b) in order]
                 + [(2 * n + 2, b) for (n, b) in order])
        waves.append(order)
    waves.reverse()
    layout = [p for wv in waves for p in wv]
    nw = [len(wv) for wv in waves]
    off = [0]
    for n in nw:
        off.append(off[-1] + n)

    ids = np.empty((len(layout), _T), np.int32)
    for q, (node, b) in enumerate(layout):
        r = b * _N + node
        ids[q] = 7 * r + 13 * np.arange(_T) + 1

    pos = {p: q for q, p in enumerate(layout)}
    post = []
    stack = [(0, False)]
    while stack:
        n, done = stack.pop()
        if done:
            post.append(n)
        else:
            stack.append((n, True))
            if 2 * n + 2 < _N:
                stack.append((2 * n + 2, False))
            if 2 * n + 1 < _N:
                stack.append((2 * n + 1, False))
    rowidx = np.empty((_N, _B), np.int32)
    for k, node in enumerate(post):
        for b in range(_B):
            rowidx[k, b] = pos[(node, b)]
    return ids, rowidx, nw, off


_IDS, _ROWIDX, _NW, _OFF = _build_schedule()
_R = _OFF[-1]


def _tree_body(pool_ref, wenc_ref, benc_ref, wc_ref, bc_ref, ws_ref, bs_ref,
               res_ref):
    enc = jnp.maximum(
        jnp.dot(pool_ref[...], wenc_ref[...],
                preferred_element_type=jnp.float32) + benc_ref[...], 0.0)
    res_ref[...] = (jnp.dot(enc, wc_ref[...],
                            preferred_element_type=jnp.float32) + bc_ref[...])
    ws = ws_ref[...]
    bs2 = 2.0 * bs_ref[...]
    for w in range(1, 7):
        o, n, po = _OFF[w], _NW[w], _OFF[w - 1]
        csum = res_ref[po:po + n, :] + res_ref[po + n:po + 2 * n, :]
        res_ref[o:o + n, :] = (
            res_ref[o:o + n, :]
            + jnp.dot(csum, ws, preferred_element_type=jnp.float32) + bs2)


def kernel(emb_table, w_enc_pad, b_enc_pad, w_c_pad, b_c_pad, w_sum_pad,
           b_sum_pad):
    poolg = jnp.mean(jnp.take(emb_table, jnp.asarray(_IDS), axis=0), axis=1)

    res = pl.pallas_call(
        _tree_body,
        out_shape=jax.ShapeDtypeStruct((_R, _FEAT), jnp.float32),
        compiler_params=pltpu.CompilerParams(vmem_limit_bytes=32 << 20),
    )(poolg, w_enc_pad, b_enc_pad, w_c_pad, b_c_pad, w_sum_pad, b_sum_pad)

    rows = jnp.maximum(jnp.take(res, jnp.asarray(_ROWIDX.reshape(-1)), axis=0),
                       0.0)
    node_stack = rows.reshape(_N, _B, _FEAT)[:, :, :_AGG]
    pooled_out = jnp.max(node_stack, axis=0)
    return node_stack, pooled_out
